# baseline (speedup 1.0000x reference)
import jax
import jax.numpy as jnp
from jax.experimental import pallas as pl
from jax.experimental.pallas import tpu as pltpu

_EPS = 1e-5
_B, _C, _H, _W = 32, 3, 400, 600
_KS = 5
_HO, _WO, _OC = _H // _KS, _W // _KS, 16
_HT = 8
_RT = _HT * _KS
_CR = _C * _RT
_NHT = _HO // _HT
_WP = 128


def _fused_kernel(x_ref, lall_ref, sw_ref, bias_ref, whp_ref, o_ref):
    def tile_body(ht, carry):
        a0, a1 = carry
        r0 = ht * _RT
        xbf = jnp.concatenate(
            [x_ref[0, c, pl.ds(r0, _RT), :].astype(jnp.bfloat16)
             for c in range(_C)], axis=0)
        g = jnp.dot(xbf, lall_ref[...],
                    preferred_element_type=jnp.float32)
        gb = g.astype(jnp.bfloat16)
        z = jnp.dot(sw_ref[0], gb[:, 0:_WP],
                    preferred_element_type=jnp.float32)
        for j in range(1, _KS):
            z = z + jnp.dot(sw_ref[j], gb[:, j * _WP:(j + 1) * _WP],
                            preferred_element_type=jnp.float32)
        z = jnp.maximum(z + bias_ref[...], 0.0)
        a0 = a0 + jnp.sum(z * whp_ref[0, ht], axis=0, keepdims=True)
        a1 = a1 + jnp.sum(z * whp_ref[1, ht], axis=0, keepdims=True)
        return (a0, a1)

    zero = jnp.zeros((1, _WP), jnp.float32)
    a0, a1 = jax.lax.fori_loop(0, _NHT, tile_body, (zero, zero))
    t0 = jnp.sum(a0)
    t1 = jnp.sum(a1)
    lane = jax.lax.broadcasted_iota(jnp.int32, (1, 1, 128), 2)
    o_ref[...] = jnp.where(lane == 0, t0, jnp.where(lane == 1, t1, 0.0))


def kernel(x, conv_w, conv_b, bn_gamma, bn_beta, bn_mean, bn_var,
           head_w, head_b):
    bn_scale = bn_gamma * jax.lax.rsqrt(bn_var + _EPS)
    w_sc = conv_w * bn_scale[:, None, None, None]
    b_eff = bn_scale * (conv_b - bn_mean) + bn_beta

    ll = jax.lax.broadcasted_iota(jnp.int32, (_W, _KS * _WP), 0)
    cc = jax.lax.broadcasted_iota(jnp.int32, (_W, _KS * _WP), 1)
    jj, ww = cc // _WP, cc % _WP
    lall = ((ww < _WO) & (ll == _KS * ww + jj)).astype(jnp.bfloat16)

    rw = jax.lax.broadcasted_iota(jnp.int32, (_KS, _OC * _HT, _CR), 1)
    cl = jax.lax.broadcasted_iota(jnp.int32, (_KS, _OC * _HT, _CR), 2)
    ja = jax.lax.broadcasted_iota(jnp.int32, (_KS, _OC * _HT, _CR), 0)
    oc_a, h_a = rw // _HT, rw % _HT
    c_a, rl_a = cl // _RT, cl % _RT
    d_a = rl_a - _KS * h_a
    valid = (d_a >= 0) & (d_a < _KS)
    flat = ((oc_a * _C + c_a) * _KS + jnp.clip(d_a, 0, _KS - 1)) * _KS + ja
    sw = jnp.where(valid, w_sc.reshape(-1)[flat], 0.0).astype(jnp.bfloat16)

    bias_mat = jnp.broadcast_to(
        jnp.repeat(b_eff, _HT)[:, None], (_OC * _HT, _WP))

    wh = head_w.reshape(2, _OC, _HO, _WO)
    wh = jnp.pad(wh, ((0, 0), (0, 0), (0, 0), (0, _WP - _WO)))
    whp = wh.reshape(2, _OC, _NHT, _HT, _WP).transpose(0, 2, 1, 3, 4)
    whp = whp.reshape(2, _NHT, _OC * _HT, _WP)

    out_pad = pl.pallas_call(
        _fused_kernel,
        out_shape=jax.ShapeDtypeStruct((_B, 1, 128), jnp.float32),
        grid_spec=pltpu.PrefetchScalarGridSpec(
            num_scalar_prefetch=0,
            grid=(_B,),
            in_specs=[
                pl.BlockSpec((1, _C, _H, _W), lambda b: (b, 0, 0, 0)),
                pl.BlockSpec((_W, _KS * _WP), lambda b: (0, 0)),
                pl.BlockSpec((_KS, _OC * _HT, _CR), lambda b: (0, 0, 0)),
                pl.BlockSpec((_OC * _HT, _WP), lambda b: (0, 0)),
                pl.BlockSpec((2, _NHT, _OC * _HT, _WP),
                             lambda b: (0, 0, 0, 0)),
            ],
            out_specs=pl.BlockSpec((1, 1, 128), lambda b: (b, 0, 0)),
        ),
        compiler_params=pltpu.CompilerParams(
            dimension_semantics=("parallel",)),
    )(x, lall, sw, bias_mat, whp)

    return out_pad[:, 0, :2] + head_b[None, :]


# speedup vs baseline: 2.0205x; 2.0205x over previous
import jax
import jax.numpy as jnp
from jax.experimental import pallas as pl
from jax.experimental.pallas import tpu as pltpu

_EPS = 1e-5
_B, _C, _H, _W = 32, 3, 400, 600
_KS = 5
_HO, _WO, _OC = _H // _KS, _W // _KS, 16
_HT = 8
_RT = _HT * _KS
_NHT = _HO // _HT
_G = 2


def _fused_kernel(x_ref, wr_ref, s_ref, b_ref, whz_ref, o_ref):
    def tile_body(i, carry):
        a0, a1 = carry
        for sub in range(_G):
            ht = i * _G + sub
            r0 = ht * _RT
            h0 = ht * _HT
            for oc in range(_OC):
                acc = x_ref[0, 0, pl.ds(r0, _RT), :] * wr_ref[oc, 0]
                for c in range(1, _C):
                    acc = (acc
                           + x_ref[0, c, pl.ds(r0, _RT), :] * wr_ref[oc, c])
                rows = jnp.dot(s_ref[...], acc,
                               preferred_element_type=jnp.float32)
                p2 = rows + pltpu.roll(rows, _W - 1, 1)
                p4 = p2 + pltpu.roll(p2, _W - 2, 1)
                s = p4 + pltpu.roll(rows, _W - 4, 1)
                r = jnp.maximum(s + b_ref[oc], 0.0)
                a0 = a0 + jnp.sum(r * whz_ref[0, oc, pl.ds(h0, _HT), :],
                                  axis=0, keepdims=True)
                a1 = a1 + jnp.sum(r * whz_ref[1, oc, pl.ds(h0, _HT), :],
                                  axis=0, keepdims=True)
        return (a0, a1)

    zero = jnp.zeros((1, _W), jnp.float32)
    a0, a1 = jax.lax.fori_loop(0, _NHT // _G, tile_body, (zero, zero))
    t0 = jnp.sum(a0)
    t1 = jnp.sum(a1)
    lane = jax.lax.broadcasted_iota(jnp.int32, (1, 1, 128), 2)
    o_ref[...] = jnp.where(lane == 0, t0, jnp.where(lane == 1, t1, 0.0))


def kernel(x, conv_w, conv_b, bn_gamma, bn_beta, bn_mean, bn_var,
           head_w, head_b):
    bn_scale = bn_gamma * jax.lax.rsqrt(bn_var + _EPS)
    w_sc = conv_w * bn_scale[:, None, None, None]
    b_eff = bn_scale * (conv_b - bn_mean) + bn_beta

    wr = jnp.tile(w_sc, (1, 1, _RT // _KS, _WO))

    row = jax.lax.broadcasted_iota(jnp.int32, (_HT, _RT), 0)
    col = jax.lax.broadcasted_iota(jnp.int32, (_HT, _RT), 1)
    s_mat = ((col >= _KS * row) & (col < _KS * row + _KS)).astype(jnp.float32)

    wh = head_w.reshape(2, _OC, _HO, _WO)
    whz = jnp.zeros((2, _OC, _HO, _W), jnp.float32)
    whz = whz.at[:, :, :, ::_KS].set(wh)

    out_pad = pl.pallas_call(
        _fused_kernel,
        out_shape=jax.ShapeDtypeStruct((_B, 1, 128), jnp.float32),
        grid_spec=pltpu.PrefetchScalarGridSpec(
            num_scalar_prefetch=0,
            grid=(_B,),
            in_specs=[
                pl.BlockSpec((1, _C, _H, _W), lambda b: (b, 0, 0, 0)),
                pl.BlockSpec((_OC, _C, _RT, _W), lambda b: (0, 0, 0, 0)),
                pl.BlockSpec((_HT, _RT), lambda b: (0, 0)),
                pl.BlockSpec(memory_space=pltpu.SMEM),
                pl.BlockSpec((2, _OC, _HO, _W), lambda b: (0, 0, 0, 0)),
            ],
            out_specs=pl.BlockSpec((1, 1, 128), lambda b: (b, 0, 0)),
        ),
        compiler_params=pltpu.CompilerParams(
            dimension_semantics=("parallel",)),
    )(x, wr, s_mat, b_eff, whz)

    return out_pad[:, 0, :2] + head_b[None, :]
